# Initial kernel scaffold; baseline (speedup 1.0000x reference)
import jax
import jax.numpy as jnp
from jax.experimental import pallas as pl


def kernel(edge_index, x, weights, lin1_w, lin1_b, conv1_w, conv1_b, mu_w, mu_b, ls_w, ls_b, out_w, out_b):
    raise NotImplementedError("write your pallas kernel here")


# speedup vs baseline: 11.8907x; 11.8907x over previous
import functools

import jax
import jax.numpy as jnp
from jax import lax
from jax.experimental import pallas as pl
from jax.experimental.pallas import tpu as pltpu
from jax.experimental.pallas import tpu_sc as plsc

N = 10000
E = 320000
NC = 2
NS = 16
C = 80
NPAD = 10240
RTILE = NPAD // NS

_mesh = lambda: plsc.VectorSubcoreMesh(core_axis_name="c", subcore_axis_name="s")


def _t1_body(x_ref, lw_ref, lb_ref, cw_ref, hw1_ref):
    h = jnp.dot(x_ref[...], lw_ref[...], preferred_element_type=jnp.float32)
    h = h + lb_ref[...][None, :]
    hw1_ref[...] = jnp.dot(h, cw_ref[...], preferred_element_type=jnp.float32)


def _t1(x, lin1_w, lin1_b, conv1_w):
    return pl.pallas_call(
        _t1_body,
        out_shape=jax.ShapeDtypeStruct((N, 2 * 128), jnp.float32),
    )(x, lin1_w, lin1_b, conv1_w)


def _t2_body(deg0_ref, deg1_ref, hw1_ref, hws_ref, dis_ref):
    deg = deg0_ref[...] + deg1_ref[...] + 1.0
    dis = lax.rsqrt(deg)
    dis_ref[...] = dis
    hws_ref[0] = hw1_ref[:, :128] * dis
    hws_ref[1] = hw1_ref[:, 128:] * dis


def _t2(deg0, deg1, hw1):
    return pl.pallas_call(
        _t2_body,
        out_shape=(
            jax.ShapeDtypeStruct((2, N, 128), jnp.float32),
            jax.ShapeDtypeStruct((N, 1), jnp.float32),
        ),
    )(deg0, deg1, hw1)


def _t3_body(acc_ref, dis_ref, hw1_ref, cb_ref, mw_ref, hws2_ref, hw2_ref):
    dis = dis_ref[...]
    acc = jnp.concatenate([acc_ref[0], acc_ref[1]], axis=1)
    h1 = jnp.maximum(dis * acc + (dis * dis) * hw1_ref[...] + cb_ref[...][None, :], 0.0)
    hw2 = jnp.dot(h1, mw_ref[...], preferred_element_type=jnp.float32)
    hw2_ref[...] = hw2
    hws2_ref[...] = hw2 * dis


def _t3(acc1, dis, hw1, conv1_b, mu_w):
    return pl.pallas_call(
        _t3_body,
        out_shape=(
            jax.ShapeDtypeStruct((N, 128), jnp.float32),
            jax.ShapeDtypeStruct((N, 128), jnp.float32),
        ),
    )(acc1, dis, hw1, conv1_b, mu_w)


def _t4_body(acc_ref, dis_ref, hw2_ref, mb_ref, ow_ref, ob_ref, p_ref, q_ref):
    dis = dis_ref[...]
    acc = acc_ref[0] + acc_ref[1]
    mu = dis * acc + (dis * dis) * hw2_ref[...] + mb_ref[...][None, :]
    ow = ow_ref[...]
    p_ref[...] = jnp.dot(mu, ow[:128, :], preferred_element_type=jnp.float32) + ob_ref[...]
    q_ref[...] = jnp.dot(mu, ow[128:, :], preferred_element_type=jnp.float32)


def _t4(acc2, dis, hw2, mu_b, out_w, out_b):
    return pl.pallas_call(
        _t4_body,
        out_shape=(
            jax.ShapeDtypeStruct((N, 1), jnp.float32),
            jax.ShapeDtypeStruct((N, 1), jnp.float32),
        ),
    )(acc2, dis, hw2, mu_b, out_w, out_b)


_EPT_DEG = E // (NC * NS)
_G = 5
_GE = _G * C


@functools.partial(
    pl.kernel,
    out_type=(
        jax.ShapeDtypeStruct((NPAD,), jnp.float32),
        jax.ShapeDtypeStruct((NPAD,), jnp.float32),
    ),
    mesh=_mesh(),
    scratch_types=[
        pltpu.VMEM((_GE,), jnp.int32),
        pltpu.VMEM((_GE,), jnp.float32),
        pltpu.VMEM((_G, C), jnp.int32),
        pltpu.VMEM_SHARED((NPAD,), jnp.float32),
    ],
)
def _deg_kernel(dst_hbm, w_hbm, zeros_hbm, deg0_hbm, deg1_hbm,
                dst_v, w_v, dst2_v, deg_sh):
    c = lax.axis_index("c")
    s = lax.axis_index("s")
    base = (c * NS + s) * _EPT_DEG
    pltpu.sync_copy(zeros_hbm.at[pl.ds(0, RTILE)], deg_sh.at[pl.ds(s * RTILE, RTILE)])
    plsc.subcore_barrier()

    def group(g, carry):
        gb = base + g * _GE
        pltpu.sync_copy(dst_hbm.at[pl.ds(gb, _GE)], dst_v)
        pltpu.sync_copy(w_hbm.at[pl.ds(gb, _GE)], w_v)

        def fill(i, carry2):
            for j in range(C // 16):
                dst2_v[i, pl.ds(j * 16, 16)] = dst_v[pl.ds(i * C + j * 16, 16)]
            return carry2

        lax.fori_loop(0, _G, fill, 0)

        def chunk(i, carry2):
            pltpu.sync_copy(w_v.at[pl.ds(i * C, C)], deg_sh.at[dst2_v.at[i]],
                            add=True)
            return carry2

        lax.fori_loop(0, _G, chunk, 0)
        return carry

    lax.fori_loop(0, _EPT_DEG // _GE, group, 0)
    plsc.subcore_barrier()

    @pl.when((s == 0) & (c == 0))
    def _():
        pltpu.sync_copy(deg_sh, deg0_hbm)

    @pl.when((s == 0) & (c == 1))
    def _():
        pltpu.sync_copy(deg_sh, deg1_hbm)


def _make_prop(feature_split):
    ept = E // NS if feature_split else E // (NC * NS)

    @functools.partial(
        pl.kernel,
        out_type=jax.ShapeDtypeStruct((2 * N, 128), jnp.float32),
        mesh=_mesh(),
        scratch_types=[
            pltpu.VMEM((_GE,), jnp.int32),
            pltpu.VMEM((_GE,), jnp.int32),
            pltpu.VMEM((_GE,), jnp.float32),
            pltpu.VMEM((_G, C), jnp.int32),
            pltpu.VMEM((C, 128), jnp.float32),
            pltpu.VMEM_SHARED((NPAD, 128), jnp.float32),
        ],
    )
    def prop(src_hbm, dst_hbm, w_hbm, tbl_hbm, zeros_hbm, out_hbm,
             src_v, dst_v, w_v, dst2_v, rows_v, acc_sh):
        c = lax.axis_index("c")
        s = lax.axis_index("s")
        base = (s * ept) if feature_split else ((c * NS + s) * ept)
        pltpu.sync_copy(zeros_hbm, acc_sh.at[pl.ds(s * RTILE, RTILE)])
        cN = c * N
        plsc.subcore_barrier()

        def group(g, carry):
            gb = base + g * _GE
            pltpu.sync_copy(src_hbm.at[pl.ds(gb, _GE)], src_v)
            pltpu.sync_copy(dst_hbm.at[pl.ds(gb, _GE)], dst_v)
            pltpu.sync_copy(w_hbm.at[pl.ds(gb, _GE)], w_v)

            def fill(i, carry2):
                for j in range(C // 16):
                    sl = pl.ds(i * C + j * 16, 16)
                    dst2_v[i, pl.ds(j * 16, 16)] = dst_v[sl]
                    if feature_split:
                        src_v[sl] = src_v[sl] + cN
                return carry2

            lax.fori_loop(0, _G, fill, 0)

            def chunk(i, carry2):
                pltpu.sync_copy(tbl_hbm.at[src_v.at[pl.ds(i * C, C)]], rows_v)

                def scale(gg, carry3):
                    w16 = w_v[pl.ds(i * C + gg * 16, 16)]
                    for lane in range(16):
                        e = gg * 16 + lane
                        wv = w16[lane]
                        for j in range(128 // 16):
                            rows_v[e, pl.ds(j * 16, 16)] = (
                                rows_v[e, pl.ds(j * 16, 16)] * wv)
                    return carry3

                lax.fori_loop(0, C // 16, scale, 0)
                pltpu.sync_copy(rows_v, acc_sh.at[dst2_v.at[i]], add=True)
                return carry2

            lax.fori_loop(0, _G, chunk, 0)
            return carry

        lax.fori_loop(0, ept // _GE, group, 0)
        plsc.subcore_barrier()

        @pl.when(s < NS - 1)
        def _():
            pltpu.sync_copy(acc_sh.at[pl.ds(s * RTILE, RTILE)],
                            out_hbm.at[pl.ds(cN + s * RTILE, RTILE)])

        @pl.when(s == NS - 1)
        def _():
            tail = N - (NS - 1) * RTILE
            pltpu.sync_copy(acc_sh.at[pl.ds((NS - 1) * RTILE, tail)],
                            out_hbm.at[pl.ds(cN + (NS - 1) * RTILE, tail)])

    return prop


_prop1 = _make_prop(True)
_prop2 = _make_prop(False)

_EPT_HEAD = E // (NC * NS)
_GE_H = 2000


@functools.partial(
    pl.kernel,
    out_type=jax.ShapeDtypeStruct((E,), jnp.float32),
    mesh=_mesh(),
    scratch_types=[
        pltpu.VMEM((N,), jnp.float32),
        pltpu.VMEM((N,), jnp.float32),
        pltpu.VMEM((_GE_H,), jnp.int32),
        pltpu.VMEM((_GE_H,), jnp.int32),
        pltpu.VMEM((_GE_H,), jnp.float32),
    ],
    compiler_params=pltpu.CompilerParams(needs_layout_passes=False),
)
def _head_kernel(src_hbm, dst_hbm, p_hbm, q_hbm, out_hbm,
                 p_v, q_v, src_v, dst_v, out_v):
    c = lax.axis_index("c")
    s = lax.axis_index("s")
    base = (c * NS + s) * _EPT_HEAD
    pltpu.sync_copy(p_hbm, p_v)
    pltpu.sync_copy(q_hbm, q_v)

    def group(g, carry):
        gb = base + g * _GE_H
        pltpu.sync_copy(src_hbm.at[pl.ds(gb, _GE_H)], src_v)
        pltpu.sync_copy(dst_hbm.at[pl.ds(gb, _GE_H)], dst_v)

        def step(i, carry2):
            sidx = src_v[pl.ds(i * 16, 16)]
            didx = dst_v[pl.ds(i * 16, 16)]
            t = plsc.load_gather(p_v, [sidx]) + plsc.load_gather(q_v, [didx])
            out_v[pl.ds(i * 16, 16)] = 1.0 / (1.0 + jnp.exp(-t))
            return carry2

        lax.fori_loop(0, _GE_H // 16, step, 0)
        pltpu.sync_copy(out_v, out_hbm.at[pl.ds(gb, _GE_H)])
        return carry

    lax.fori_loop(0, _EPT_HEAD // _GE_H, group, 0)


def kernel(edge_index, x, weights, lin1_w, lin1_b, conv1_w, conv1_b,
           mu_w, mu_b, ls_w, ls_b, out_w, out_b):
    src = edge_index[0]
    dst = edge_index[1]
    zrpt = jnp.zeros((RTILE,), jnp.float32)
    z128 = jnp.zeros((RTILE, 128), jnp.float32)

    hw1 = _t1(x, lin1_w, lin1_b, conv1_w)
    deg0, deg1 = _deg_kernel(dst, weights, zrpt)
    hws1, dis = _t2(deg0[:N].reshape(N, 1), deg1[:N].reshape(N, 1), hw1)
    acc1 = _prop1(src, dst, weights, hws1.reshape(2 * N, 128), z128)
    hws2, hw2 = _t3(acc1.reshape(2, N, 128), dis, hw1, conv1_b, mu_w)
    acc2 = _prop2(src, dst, weights, hws2, z128)
    p, q = _t4(acc2.reshape(2, N, 128), dis, hw2, mu_b, out_w, out_b)
    out = _head_kernel(src, dst, p.reshape(N), q.reshape(N))
    return out


# speedup vs baseline: 17.0586x; 1.4346x over previous
import functools

import jax
import jax.numpy as jnp
from jax import lax
from jax.experimental import pallas as pl
from jax.experimental.pallas import tpu as pltpu
from jax.experimental.pallas import tpu_sc as plsc

N = 10000
E = 320000
NC = 2
NS = 16
C = 80
NPAD = 10240
RTILE = NPAD // NS

_mesh = lambda: plsc.VectorSubcoreMesh(core_axis_name="c", subcore_axis_name="s")


def _t1_body(x_ref, lw_ref, lb_ref, cw_ref, hw1_ref):
    h = jnp.dot(x_ref[...], lw_ref[...], preferred_element_type=jnp.float32)
    h = h + lb_ref[...][None, :]
    hw1_ref[...] = jnp.dot(h, cw_ref[...], preferred_element_type=jnp.float32)


def _t1(x, lin1_w, lin1_b, conv1_w):
    return pl.pallas_call(
        _t1_body,
        out_shape=jax.ShapeDtypeStruct((N, 2 * 128), jnp.float32),
    )(x, lin1_w, lin1_b, conv1_w)


def _t2_body(deg0_ref, deg1_ref, hw1_ref, hws_ref, dis_ref):
    deg = deg0_ref[...] + deg1_ref[...] + 1.0
    dis = lax.rsqrt(deg)
    dis_ref[...] = dis
    hws_ref[0] = hw1_ref[:, :128] * dis
    hws_ref[1] = hw1_ref[:, 128:] * dis


def _t2(deg0, deg1, hw1):
    return pl.pallas_call(
        _t2_body,
        out_shape=(
            jax.ShapeDtypeStruct((2, N, 128), jnp.float32),
            jax.ShapeDtypeStruct((N, 1), jnp.float32),
        ),
    )(deg0, deg1, hw1)


def _t3_body(acc_ref, dis_ref, hw1_ref, cb_ref, mw_ref, hws2_ref, hw2_ref):
    dis = dis_ref[...]
    acc = jnp.concatenate([acc_ref[0], acc_ref[1]], axis=1)
    h1 = jnp.maximum(dis * acc + (dis * dis) * hw1_ref[...] + cb_ref[...][None, :], 0.0)
    hw2 = jnp.dot(h1, mw_ref[...], preferred_element_type=jnp.float32)
    hw2_ref[...] = hw2
    hws2_ref[...] = hw2 * dis


def _t3(acc1, dis, hw1, conv1_b, mu_w):
    return pl.pallas_call(
        _t3_body,
        out_shape=(
            jax.ShapeDtypeStruct((N, 128), jnp.float32),
            jax.ShapeDtypeStruct((N, 128), jnp.float32),
        ),
    )(acc1, dis, hw1, conv1_b, mu_w)


def _t4_body(acc_ref, dis_ref, hw2_ref, mb_ref, ow_ref, ob_ref, p_ref, q_ref):
    dis = dis_ref[...]
    acc = acc_ref[0] + acc_ref[1]
    mu = dis * acc + (dis * dis) * hw2_ref[...] + mb_ref[...][None, :]
    ow = ow_ref[...]
    p_ref[...] = jnp.dot(mu, ow[:128, :], preferred_element_type=jnp.float32) + ob_ref[...]
    q_ref[...] = jnp.dot(mu, ow[128:, :], preferred_element_type=jnp.float32)


def _t4(acc2, dis, hw2, mu_b, out_w, out_b):
    return pl.pallas_call(
        _t4_body,
        out_shape=(
            jax.ShapeDtypeStruct((N, 1), jnp.float32),
            jax.ShapeDtypeStruct((N, 1), jnp.float32),
        ),
    )(acc2, dis, hw2, mu_b, out_w, out_b)


_SLOTS = 5
_CP = 40
_CD = 80

_EPT_DEG = E // (NC * NS)
_NCH_DEG = _EPT_DEG // _CD


def _deg_scratch():
    out = []
    for _ in range(_SLOTS):
        out.append(pltpu.VMEM((_CD,), jnp.int32))
    for _ in range(_SLOTS):
        out.append(pltpu.VMEM((_CD,), jnp.float32))
    out.append(pltpu.VMEM_SHARED((NPAD,), jnp.float32))
    out.append(pltpu.SemaphoreType.DMA((_SLOTS,)))
    out.append(pltpu.SemaphoreType.DMA((_SLOTS,)))
    return out


@functools.partial(
    pl.kernel,
    out_type=(
        jax.ShapeDtypeStruct((NPAD,), jnp.float32),
        jax.ShapeDtypeStruct((NPAD,), jnp.float32),
    ),
    mesh=_mesh(),
    scratch_types=_deg_scratch(),
)
def _deg_kernel(dst_hbm, w_hbm, zeros_hbm, deg0_hbm, deg1_hbm, *r):
    dsts = r[0:_SLOTS]
    ws = r[_SLOTS:2 * _SLOTS]
    deg_sh, esem, ssem = r[2 * _SLOTS], r[2 * _SLOTS + 1], r[2 * _SLOTS + 2]
    c = lax.axis_index("c")
    s = lax.axis_index("s")
    base = (c * NS + s) * _EPT_DEG
    pltpu.sync_copy(zeros_hbm.at[pl.ds(0, RTILE)], deg_sh.at[pl.ds(s * RTILE, RTILE)])
    plsc.subcore_barrier()

    def el(i, b):
        eb = base + i * _CD
        pltpu.async_copy(dst_hbm.at[pl.ds(eb, _CD)], dsts[b], esem.at[b])
        pltpu.async_copy(w_hbm.at[pl.ds(eb, _CD)], ws[b], esem.at[b])

    def we(i, b):
        eb = base + i * _CD
        pltpu.make_async_copy(dst_hbm.at[pl.ds(eb, _CD)], dsts[b], esem.at[b]).wait()
        pltpu.make_async_copy(w_hbm.at[pl.ds(eb, _CD)], ws[b], esem.at[b]).wait()

    def ss(i, b):
        pltpu.async_copy(ws[b], deg_sh.at[dsts[b]], ssem.at[b], add=True)

    def ws_(i, b):
        pltpu.make_async_copy(ws[b], deg_sh.at[dsts[b]], ssem.at[b]).wait()

    el(0, 0)
    el(1, 1)

    def five(p, carry):
        for b in range(_SLOTS):
            i = p * _SLOTS + b
            we(i, b)
            ss(i, b)

            @pl.when(i - 2 >= 0)
            def _():
                ws_(i - 2, (b + 3) % _SLOTS)

            @pl.when(i + 2 < _NCH_DEG)
            def _():
                el(i + 2, (b + 2) % _SLOTS)
        return carry

    lax.fori_loop(0, _NCH_DEG // _SLOTS, five, 0)
    ws_(_NCH_DEG - 2, (_NCH_DEG - 2) % _SLOTS)
    ws_(_NCH_DEG - 1, (_NCH_DEG - 1) % _SLOTS)
    plsc.subcore_barrier()

    @pl.when((s == 0) & (c == 0))
    def _():
        pltpu.sync_copy(deg_sh, deg0_hbm)

    @pl.when((s == 0) & (c == 1))
    def _():
        pltpu.sync_copy(deg_sh, deg1_hbm)


def _prop_scratch():
    out = []
    for _ in range(_SLOTS):
        out.append(pltpu.VMEM((48,), jnp.int32))
    for _ in range(_SLOTS):
        out.append(pltpu.VMEM((_CP,), jnp.int32))
    for _ in range(_SLOTS):
        out.append(pltpu.VMEM((48,), jnp.float32))
    for _ in range(_SLOTS):
        out.append(pltpu.VMEM((_CP, 128), jnp.float32))
    out.append(pltpu.VMEM_SHARED((NPAD, 128), jnp.float32))
    out.append(pltpu.SemaphoreType.DMA((_SLOTS,)))
    out.append(pltpu.SemaphoreType.DMA((_SLOTS,)))
    out.append(pltpu.SemaphoreType.DMA((_SLOTS,)))
    return out


def _make_prop(feature_split):
    ept = E // NS if feature_split else E // (NC * NS)
    nch = ept // _CP

    @functools.partial(
        pl.kernel,
        out_type=jax.ShapeDtypeStruct((2 * N, 128), jnp.float32),
        mesh=_mesh(),
        scratch_types=_prop_scratch(),
    )
    def prop(src_hbm, dst_hbm, w_hbm, tbl_hbm, zeros_hbm, out_hbm, *r):
        srcs = r[0:_SLOTS]
        dsts = r[_SLOTS:2 * _SLOTS]
        wbs = r[2 * _SLOTS:3 * _SLOTS]
        rows = r[3 * _SLOTS:4 * _SLOTS]
        acc_sh = r[4 * _SLOTS]
        esem, gsem, ssem = r[4 * _SLOTS + 1], r[4 * _SLOTS + 2], r[4 * _SLOTS + 3]
        c = lax.axis_index("c")
        s = lax.axis_index("s")
        base = (s * ept) if feature_split else ((c * NS + s) * ept)
        pltpu.sync_copy(zeros_hbm, acc_sh.at[pl.ds(s * RTILE, RTILE)])
        cN = c * N
        plsc.subcore_barrier()

        def el(i, b):
            eb = base + i * _CP
            pltpu.async_copy(src_hbm.at[pl.ds(eb, _CP)], srcs[b].at[pl.ds(0, _CP)],
                             esem.at[b])
            pltpu.async_copy(dst_hbm.at[pl.ds(eb, _CP)], dsts[b], esem.at[b])
            pltpu.async_copy(w_hbm.at[pl.ds(eb, _CP)], wbs[b].at[pl.ds(0, _CP)],
                             esem.at[b])

        def we(i, b):
            eb = base + i * _CP
            pltpu.make_async_copy(src_hbm.at[pl.ds(eb, _CP)],
                                  srcs[b].at[pl.ds(0, _CP)], esem.at[b]).wait()
            pltpu.make_async_copy(dst_hbm.at[pl.ds(eb, _CP)], dsts[b],
                                  esem.at[b]).wait()
            pltpu.make_async_copy(w_hbm.at[pl.ds(eb, _CP)],
                                  wbs[b].at[pl.ds(0, _CP)], esem.at[b]).wait()

        def adj(b):
            if feature_split:
                for j in range(3):
                    srcs[b][pl.ds(j * 16, 16)] = srcs[b][pl.ds(j * 16, 16)] + cN

        def sg(i, b):
            pltpu.async_copy(tbl_hbm.at[srcs[b].at[pl.ds(0, _CP)]], rows[b],
                             gsem.at[b])

        def wg(i, b):
            pltpu.make_async_copy(tbl_hbm.at[srcs[b].at[pl.ds(0, _CP)]], rows[b],
                                  gsem.at[b]).wait()

        def ss(i, b):
            pltpu.async_copy(rows[b], acc_sh.at[dsts[b]], ssem.at[b], add=True)

        def ws_(i, b):
            pltpu.make_async_copy(rows[b], acc_sh.at[dsts[b]], ssem.at[b]).wait()

        def scale(b):
            for g in range(3):
                w16 = wbs[b][pl.ds(g * 16, 16)]
                for lane in range(16 if g < 2 else 8):
                    e = g * 16 + lane
                    wv = w16[lane]
                    for j in range(8):
                        rows[b][e, pl.ds(j * 16, 16)] = (
                            rows[b][e, pl.ds(j * 16, 16)] * wv)

        el(0, 0)
        el(1, 1)
        el(2, 2)
        we(0, 0); adj(0); sg(0, 0)
        we(1, 1); adj(1); sg(1, 1)

        def five(p, carry):
            for b in range(_SLOTS):
                i = p * _SLOTS + b
                wg(i, b)
                scale(b)
                ss(i, b)
                b3 = (b + 3) % _SLOTS

                @pl.when(i - 2 >= 0)
                def _():
                    ws_(i - 2, b3)

                @pl.when(i + 3 < nch)
                def _():
                    el(i + 3, b3)

                b2 = (b + 2) % _SLOTS

                @pl.when(i + 2 < nch)
                def _():
                    we(i + 2, b2)
                    adj(b2)
                    sg(i + 2, b2)
            return carry

        lax.fori_loop(0, nch // _SLOTS, five, 0)
        ws_(nch - 2, (nch - 2) % _SLOTS)
        ws_(nch - 1, (nch - 1) % _SLOTS)
        plsc.subcore_barrier()

        @pl.when(s < NS - 1)
        def _():
            pltpu.sync_copy(acc_sh.at[pl.ds(s * RTILE, RTILE)],
                            out_hbm.at[pl.ds(cN + s * RTILE, RTILE)])

        @pl.when(s == NS - 1)
        def _():
            tail = N - (NS - 1) * RTILE
            pltpu.sync_copy(acc_sh.at[pl.ds((NS - 1) * RTILE, tail)],
                            out_hbm.at[pl.ds(cN + (NS - 1) * RTILE, tail)])

    return prop


_prop1 = _make_prop(True)
_prop2 = _make_prop(False)

_EPT_HEAD = E // (NC * NS)
_GE_H = 2000


@functools.partial(
    pl.kernel,
    out_type=jax.ShapeDtypeStruct((E,), jnp.float32),
    mesh=_mesh(),
    scratch_types=[
        pltpu.VMEM((N,), jnp.float32),
        pltpu.VMEM((N,), jnp.float32),
        pltpu.VMEM((_GE_H,), jnp.int32),
        pltpu.VMEM((_GE_H,), jnp.int32),
        pltpu.VMEM((_GE_H,), jnp.float32),
    ],
    compiler_params=pltpu.CompilerParams(needs_layout_passes=False),
)
def _head_kernel(src_hbm, dst_hbm, p_hbm, q_hbm, out_hbm,
                 p_v, q_v, src_v, dst_v, out_v):
    c = lax.axis_index("c")
    s = lax.axis_index("s")
    base = (c * NS + s) * _EPT_HEAD
    pltpu.sync_copy(p_hbm, p_v)
    pltpu.sync_copy(q_hbm, q_v)

    def group(g, carry):
        gb = base + g * _GE_H
        pltpu.sync_copy(src_hbm.at[pl.ds(gb, _GE_H)], src_v)
        pltpu.sync_copy(dst_hbm.at[pl.ds(gb, _GE_H)], dst_v)

        def step(i, carry2):
            sidx = src_v[pl.ds(i * 16, 16)]
            didx = dst_v[pl.ds(i * 16, 16)]
            t = plsc.load_gather(p_v, [sidx]) + plsc.load_gather(q_v, [didx])
            out_v[pl.ds(i * 16, 16)] = 1.0 / (1.0 + jnp.exp(-t))
            return carry2

        lax.fori_loop(0, _GE_H // 16, step, 0)
        pltpu.sync_copy(out_v, out_hbm.at[pl.ds(gb, _GE_H)])
        return carry

    lax.fori_loop(0, _EPT_HEAD // _GE_H, group, 0)


def kernel(edge_index, x, weights, lin1_w, lin1_b, conv1_w, conv1_b,
           mu_w, mu_b, ls_w, ls_b, out_w, out_b):
    src = edge_index[0]
    dst = edge_index[1]
    zrpt = jnp.zeros((RTILE,), jnp.float32)
    z128 = jnp.zeros((RTILE, 128), jnp.float32)

    hw1 = _t1(x, lin1_w, lin1_b, conv1_w)
    deg0, deg1 = _deg_kernel(dst, weights, zrpt)
    hws1, dis = _t2(deg0[:N].reshape(N, 1), deg1[:N].reshape(N, 1), hw1)
    acc1 = _prop1(src, dst, weights, hws1.reshape(2 * N, 128), z128)
    hws2, hw2 = _t3(acc1.reshape(2, N, 128), dis, hw1, conv1_b, mu_w)
    acc2 = _prop2(src, dst, weights, hws2, z128)
    p, q = _t4(acc2.reshape(2, N, 128), dis, hw2, mu_b, out_w, out_b)
    out = _head_kernel(src, dst, p.reshape(N), q.reshape(N))
    return out


# speedup vs baseline: 18.6811x; 1.0951x over previous
import functools

import jax
import jax.numpy as jnp
from jax import lax
from jax.experimental import pallas as pl
from jax.experimental.pallas import tpu as pltpu
from jax.experimental.pallas import tpu_sc as plsc

N = 10000
E = 320000
NC = 2
NS = 16
C = 80
NPAD = 10240
RTILE = NPAD // NS

_mesh = lambda: plsc.VectorSubcoreMesh(core_axis_name="c", subcore_axis_name="s")


def _t1_body(x_ref, lw_ref, lb_ref, cw_ref, hw1_ref):
    h = jnp.dot(x_ref[...], lw_ref[...], preferred_element_type=jnp.float32)
    h = h + lb_ref[...][None, :]
    hw1_ref[...] = jnp.dot(h, cw_ref[...], preferred_element_type=jnp.float32)


def _t1(x, lin1_w, lin1_b, conv1_w):
    return pl.pallas_call(
        _t1_body,
        out_shape=jax.ShapeDtypeStruct((N, 2 * 128), jnp.float32),
    )(x, lin1_w, lin1_b, conv1_w)


def _t2_body(deg0_ref, deg1_ref, hw1_ref, hws_ref, dis_ref):
    deg = deg0_ref[...] + deg1_ref[...] + 1.0
    dis = lax.rsqrt(deg)
    dis_ref[...] = dis
    hws_ref[0] = hw1_ref[:, :128] * dis
    hws_ref[1] = hw1_ref[:, 128:] * dis


def _t2(deg0, deg1, hw1):
    return pl.pallas_call(
        _t2_body,
        out_shape=(
            jax.ShapeDtypeStruct((2, N, 128), jnp.float32),
            jax.ShapeDtypeStruct((N, 1), jnp.float32),
        ),
    )(deg0, deg1, hw1)


def _t3_body(acc_ref, dis_ref, hw1_ref, cb_ref, mw_ref, hws2_ref, hw2_ref):
    dis = dis_ref[...]
    acc = jnp.concatenate([acc_ref[0], acc_ref[1]], axis=1)
    h1 = jnp.maximum(dis * acc + (dis * dis) * hw1_ref[...] + cb_ref[...][None, :], 0.0)
    hw2 = jnp.dot(h1, mw_ref[...], preferred_element_type=jnp.float32)
    hw2_ref[...] = hw2
    hws2_ref[...] = hw2 * dis


def _t3(acc1, dis, hw1, conv1_b, mu_w):
    return pl.pallas_call(
        _t3_body,
        out_shape=(
            jax.ShapeDtypeStruct((N, 128), jnp.float32),
            jax.ShapeDtypeStruct((N, 128), jnp.float32),
        ),
    )(acc1, dis, hw1, conv1_b, mu_w)


def _t4_body(acc_ref, dis_ref, hw2_ref, mb_ref, ow_ref, ob_ref, p_ref, q_ref):
    dis = dis_ref[...]
    acc = acc_ref[0] + acc_ref[1]
    mu = dis * acc + (dis * dis) * hw2_ref[...] + mb_ref[...][None, :]
    ow = ow_ref[...]
    p_ref[...] = jnp.dot(mu, ow[:128, :], preferred_element_type=jnp.float32) + ob_ref[...]
    q_ref[...] = jnp.dot(mu, ow[128:, :], preferred_element_type=jnp.float32)


def _t4(acc2, dis, hw2, mu_b, out_w, out_b):
    return pl.pallas_call(
        _t4_body,
        out_shape=(
            jax.ShapeDtypeStruct((N, 1), jnp.float32),
            jax.ShapeDtypeStruct((N, 1), jnp.float32),
        ),
    )(acc2, dis, hw2, mu_b, out_w, out_b)


_SLOTS = 5
_CP = 40
_CD = 80

_EPT_DEG = E // (NC * NS)
_NCH_DEG = _EPT_DEG // _CD


def _deg_scratch():
    out = []
    for _ in range(_SLOTS):
        out.append(pltpu.VMEM((_CD,), jnp.int32))
    for _ in range(_SLOTS):
        out.append(pltpu.VMEM((_CD,), jnp.float32))
    out.append(pltpu.VMEM_SHARED((NPAD,), jnp.float32))
    out.append(pltpu.SemaphoreType.DMA((_SLOTS,)))
    out.append(pltpu.SemaphoreType.DMA((_SLOTS,)))
    return out


@functools.partial(
    pl.kernel,
    out_type=(
        jax.ShapeDtypeStruct((NPAD,), jnp.float32),
        jax.ShapeDtypeStruct((NPAD,), jnp.float32),
    ),
    mesh=_mesh(),
    scratch_types=_deg_scratch(),
)
def _deg_kernel(dst_hbm, w_hbm, zeros_hbm, zi_hbm, zf_hbm, deg0_hbm, deg1_hbm, *r):
    dsts = r[0:_SLOTS]
    ws = r[_SLOTS:2 * _SLOTS]
    deg_sh, esem, ssem = r[2 * _SLOTS], r[2 * _SLOTS + 1], r[2 * _SLOTS + 2]
    c = lax.axis_index("c")
    s = lax.axis_index("s")
    base = (c * NS + s) * _EPT_DEG
    pltpu.sync_copy(zeros_hbm.at[pl.ds(0, RTILE)], deg_sh.at[pl.ds(s * RTILE, RTILE)])
    plsc.subcore_barrier()

    def el(i, b):
        eb = base + i * _CD
        pltpu.async_copy(dst_hbm.at[pl.ds(eb, _CD)], dsts[b], esem.at[b])
        pltpu.async_copy(w_hbm.at[pl.ds(eb, _CD)], ws[b], esem.at[b])

    def we(i, b):
        eb = base + i * _CD
        pltpu.make_async_copy(dst_hbm.at[pl.ds(eb, _CD)], dsts[b], esem.at[b]).wait()
        pltpu.make_async_copy(w_hbm.at[pl.ds(eb, _CD)], ws[b], esem.at[b]).wait()

    def ss(i, b):
        pltpu.async_copy(ws[b], deg_sh.at[dsts[b]], ssem.at[b], add=True)

    def ws_(i, b):
        pltpu.make_async_copy(ws[b], deg_sh.at[dsts[b]], ssem.at[b]).wait()

    for b in (3, 4):
        pltpu.sync_copy(zi_hbm, dsts[b])
        pltpu.sync_copy(zf_hbm, ws[b])
        ss(-1, b)
    el(0, 0)
    el(1, 1)

    def five(p, carry):
        for b in range(_SLOTS):
            i = p * _SLOTS + b
            we(i, b)
            ss(i, b)
            ws_(i - 2, (b + 3) % _SLOTS)
            el(i + 2, (b + 2) % _SLOTS)
        return carry

    lax.fori_loop(0, _NCH_DEG // _SLOTS, five, 0)
    ws_(_NCH_DEG - 2, (_NCH_DEG - 2) % _SLOTS)
    ws_(_NCH_DEG - 1, (_NCH_DEG - 1) % _SLOTS)
    we(_NCH_DEG, _NCH_DEG % _SLOTS)
    we(_NCH_DEG + 1, (_NCH_DEG + 1) % _SLOTS)
    plsc.subcore_barrier()

    @pl.when((s == 0) & (c == 0))
    def _():
        pltpu.sync_copy(deg_sh, deg0_hbm)

    @pl.when((s == 0) & (c == 1))
    def _():
        pltpu.sync_copy(deg_sh, deg1_hbm)


def _prop_scratch():
    out = []
    for _ in range(_SLOTS):
        out.append(pltpu.VMEM((48,), jnp.int32))
    for _ in range(_SLOTS):
        out.append(pltpu.VMEM((_CP,), jnp.int32))
    for _ in range(_SLOTS):
        out.append(pltpu.VMEM((48,), jnp.float32))
    for _ in range(_SLOTS):
        out.append(pltpu.VMEM((_CP, 128), jnp.float32))
    out.append(pltpu.VMEM_SHARED((NPAD, 128), jnp.float32))
    out.append(pltpu.SemaphoreType.DMA((_SLOTS,)))
    out.append(pltpu.SemaphoreType.DMA((_SLOTS,)))
    out.append(pltpu.SemaphoreType.DMA((_SLOTS,)))
    return out


def _make_prop(feature_split):
    ept = E // NS if feature_split else E // (NC * NS)
    nch = ept // _CP

    @functools.partial(
        pl.kernel,
        out_type=jax.ShapeDtypeStruct((2 * N, 128), jnp.float32),
        mesh=_mesh(),
        scratch_types=_prop_scratch(),
    )
    def prop(src_hbm, dst_hbm, w_hbm, tbl_hbm, zeros_hbm, zi_hbm, out_hbm, *r):
        srcs = r[0:_SLOTS]
        dsts = r[_SLOTS:2 * _SLOTS]
        wbs = r[2 * _SLOTS:3 * _SLOTS]
        rows = r[3 * _SLOTS:4 * _SLOTS]
        acc_sh = r[4 * _SLOTS]
        esem, gsem, ssem = r[4 * _SLOTS + 1], r[4 * _SLOTS + 2], r[4 * _SLOTS + 3]
        c = lax.axis_index("c")
        s = lax.axis_index("s")
        base = (s * ept) if feature_split else ((c * NS + s) * ept)
        pltpu.sync_copy(zeros_hbm, acc_sh.at[pl.ds(s * RTILE, RTILE)])
        cN = c * N
        plsc.subcore_barrier()

        def el(i, b):
            eb = base + i * _CP
            pltpu.async_copy(src_hbm.at[pl.ds(eb, _CP)], srcs[b].at[pl.ds(0, _CP)],
                             esem.at[b])
            pltpu.async_copy(dst_hbm.at[pl.ds(eb, _CP)], dsts[b], esem.at[b])
            pltpu.async_copy(w_hbm.at[pl.ds(eb, _CP)], wbs[b].at[pl.ds(0, _CP)],
                             esem.at[b])

        def we(i, b):
            eb = base + i * _CP
            pltpu.make_async_copy(src_hbm.at[pl.ds(eb, _CP)],
                                  srcs[b].at[pl.ds(0, _CP)], esem.at[b]).wait()
            pltpu.make_async_copy(dst_hbm.at[pl.ds(eb, _CP)], dsts[b],
                                  esem.at[b]).wait()
            pltpu.make_async_copy(w_hbm.at[pl.ds(eb, _CP)],
                                  wbs[b].at[pl.ds(0, _CP)], esem.at[b]).wait()

        def adj(b):
            if feature_split:
                for j in range(3):
                    srcs[b][pl.ds(j * 16, 16)] = srcs[b][pl.ds(j * 16, 16)] + cN

        def sg(i, b):
            pltpu.async_copy(tbl_hbm.at[srcs[b].at[pl.ds(0, _CP)]], rows[b],
                             gsem.at[b])

        def wg(i, b):
            pltpu.make_async_copy(tbl_hbm.at[srcs[b].at[pl.ds(0, _CP)]], rows[b],
                                  gsem.at[b]).wait()

        def ss(i, b):
            pltpu.async_copy(rows[b], acc_sh.at[dsts[b]], ssem.at[b], add=True)

        def ws_(i, b):
            pltpu.make_async_copy(rows[b], acc_sh.at[dsts[b]], ssem.at[b]).wait()

        def scale(b):
            for g in range(3):
                w16 = wbs[b][pl.ds(g * 16, 16)]
                for lane in range(16 if g < 2 else 8):
                    e = g * 16 + lane
                    wv = w16[lane]
                    for j in range(8):
                        rows[b][e, pl.ds(j * 16, 16)] = (
                            rows[b][e, pl.ds(j * 16, 16)] * wv)

        for b in (3, 4):
            pltpu.sync_copy(zeros_hbm.at[pl.ds(0, _CP)], rows[b])
            pltpu.sync_copy(zi_hbm, dsts[b])
            ss(-1, b)
        el(0, 0)
        el(1, 1)
        el(2, 2)
        we(0, 0); adj(0); sg(0, 0)
        we(1, 1); adj(1); sg(1, 1)

        def five(p, carry):
            for b in range(_SLOTS):
                i = p * _SLOTS + b
                b2 = (b + 2) % _SLOTS
                b3 = (b + 3) % _SLOTS
                wg(i, b)
                ws_(i - 2, b3)
                el(i + 3, b3)
                we(i + 2, b2)
                adj(b2)
                sg(i + 2, b2)
                scale(b)
                ss(i, b)
            return carry

        lax.fori_loop(0, nch // _SLOTS, five, 0)
        ws_(nch - 2, (nch - 2) % _SLOTS)
        ws_(nch - 1, (nch - 1) % _SLOTS)
        wg(nch, nch % _SLOTS)
        wg(nch + 1, (nch + 1) % _SLOTS)
        we(nch + 2, (nch + 2) % _SLOTS)
        plsc.subcore_barrier()

        @pl.when(s < NS - 1)
        def _():
            pltpu.sync_copy(acc_sh.at[pl.ds(s * RTILE, RTILE)],
                            out_hbm.at[pl.ds(cN + s * RTILE, RTILE)])

        @pl.when(s == NS - 1)
        def _():
            tail = N - (NS - 1) * RTILE
            pltpu.sync_copy(acc_sh.at[pl.ds((NS - 1) * RTILE, tail)],
                            out_hbm.at[pl.ds(cN + (NS - 1) * RTILE, tail)])

    return prop


_prop1 = _make_prop(True)
_prop2 = _make_prop(False)

_EPT_HEAD = E // (NC * NS)
_GE_H = 2000


@functools.partial(
    pl.kernel,
    out_type=jax.ShapeDtypeStruct((E,), jnp.float32),
    mesh=_mesh(),
    scratch_types=[
        pltpu.VMEM((N,), jnp.float32),
        pltpu.VMEM((N,), jnp.float32),
        pltpu.VMEM((_GE_H,), jnp.int32),
        pltpu.VMEM((_GE_H,), jnp.int32),
        pltpu.VMEM((_GE_H,), jnp.float32),
    ],
    compiler_params=pltpu.CompilerParams(needs_layout_passes=False),
)
def _head_kernel(src_hbm, dst_hbm, p_hbm, q_hbm, out_hbm,
                 p_v, q_v, src_v, dst_v, out_v):
    c = lax.axis_index("c")
    s = lax.axis_index("s")
    base = (c * NS + s) * _EPT_HEAD
    pltpu.sync_copy(p_hbm, p_v)
    pltpu.sync_copy(q_hbm, q_v)

    def group(g, carry):
        gb = base + g * _GE_H
        pltpu.sync_copy(src_hbm.at[pl.ds(gb, _GE_H)], src_v)
        pltpu.sync_copy(dst_hbm.at[pl.ds(gb, _GE_H)], dst_v)

        def step(i, carry2):
            sidx = src_v[pl.ds(i * 16, 16)]
            didx = dst_v[pl.ds(i * 16, 16)]
            t = plsc.load_gather(p_v, [sidx]) + plsc.load_gather(q_v, [didx])
            out_v[pl.ds(i * 16, 16)] = 1.0 / (1.0 + jnp.exp(-t))
            return carry2

        lax.fori_loop(0, _GE_H // 16, step, 0)
        pltpu.sync_copy(out_v, out_hbm.at[pl.ds(gb, _GE_H)])
        return carry

    lax.fori_loop(0, _EPT_HEAD // _GE_H, group, 0)


def kernel(edge_index, x, weights, lin1_w, lin1_b, conv1_w, conv1_b,
           mu_w, mu_b, ls_w, ls_b, out_w, out_b):
    src = edge_index[0]
    dst = edge_index[1]
    pad = 256
    srcp = jnp.concatenate([src, jnp.zeros((pad,), jnp.int32)])
    dstp = jnp.concatenate([dst, jnp.zeros((pad,), jnp.int32)])
    wp = jnp.concatenate([weights, jnp.zeros((pad,), jnp.float32)])
    zrt = jnp.zeros((RTILE,), jnp.float32)
    z128 = jnp.zeros((RTILE, 128), jnp.float32)
    zi40 = jnp.zeros((_CP,), jnp.int32)
    zi80 = jnp.zeros((_CD,), jnp.int32)
    zf80 = jnp.zeros((_CD,), jnp.float32)

    hw1 = _t1(x, lin1_w, lin1_b, conv1_w)
    deg0, deg1 = _deg_kernel(dstp, wp, zrt, zi80, zf80)
    hws1, dis = _t2(deg0[:N].reshape(N, 1), deg1[:N].reshape(N, 1), hw1)
    acc1 = _prop1(srcp, dstp, wp, hws1.reshape(2 * N, 128), z128, zi40)
    hws2, hw2 = _t3(acc1.reshape(2, N, 128), dis, hw1, conv1_b, mu_w)
    acc2 = _prop2(srcp, dstp, wp, hws2, z128, zi40)
    p, q = _t4(acc2.reshape(2, N, 128), dis, hw2, mu_b, out_w, out_b)
    out = _head_kernel(src, dst, p.reshape(N), q.reshape(N))
    return out


# speedup vs baseline: 18.7325x; 1.0028x over previous
import functools

import jax
import jax.numpy as jnp
from jax import lax
from jax.experimental import pallas as pl
from jax.experimental.pallas import tpu as pltpu
from jax.experimental.pallas import tpu_sc as plsc

N = 10000
E = 320000
NC = 2
NS = 16
C = 80
NPAD = 10240
RTILE = NPAD // NS

_mesh = lambda: plsc.VectorSubcoreMesh(core_axis_name="c", subcore_axis_name="s")


def _t1_body(x_ref, lw_ref, lb_ref, cw_ref, hw1_ref):
    h = jnp.dot(x_ref[...], lw_ref[...], preferred_element_type=jnp.float32)
    h = h + lb_ref[...][None, :]
    hw1_ref[...] = jnp.dot(h, cw_ref[...], preferred_element_type=jnp.float32)


def _t1(x, lin1_w, lin1_b, conv1_w):
    return pl.pallas_call(
        _t1_body,
        out_shape=jax.ShapeDtypeStruct((N, 2 * 128), jnp.float32),
    )(x, lin1_w, lin1_b, conv1_w)


def _t2_body(deg0_ref, deg1_ref, hw1_ref, hws_ref, dis_ref):
    deg = deg0_ref[...] + deg1_ref[...] + 1.0
    dis = lax.rsqrt(deg)
    dis_ref[...] = dis
    hws_ref[0] = hw1_ref[:, :128] * dis
    hws_ref[1] = hw1_ref[:, 128:] * dis


def _t2(deg0, deg1, hw1):
    return pl.pallas_call(
        _t2_body,
        out_shape=(
            jax.ShapeDtypeStruct((2, N, 128), jnp.float32),
            jax.ShapeDtypeStruct((N, 1), jnp.float32),
        ),
    )(deg0, deg1, hw1)


def _t3_body(acc_ref, dis_ref, hw1_ref, cb_ref, mw_ref, hws2_ref, hw2_ref):
    dis = dis_ref[...]
    acc = jnp.concatenate([acc_ref[0], acc_ref[1]], axis=1)
    h1 = jnp.maximum(dis * acc + (dis * dis) * hw1_ref[...] + cb_ref[...][None, :], 0.0)
    hw2 = jnp.dot(h1, mw_ref[...], preferred_element_type=jnp.float32)
    hw2_ref[...] = hw2
    hws2_ref[...] = hw2 * dis


def _t3(acc1, dis, hw1, conv1_b, mu_w):
    return pl.pallas_call(
        _t3_body,
        out_shape=(
            jax.ShapeDtypeStruct((N, 128), jnp.float32),
            jax.ShapeDtypeStruct((N, 128), jnp.float32),
        ),
    )(acc1, dis, hw1, conv1_b, mu_w)


def _t4_body(acc_ref, dis_ref, hw2_ref, mb_ref, ow_ref, ob_ref, p_ref, q_ref):
    dis = dis_ref[...]
    acc = acc_ref[0] + acc_ref[1]
    mu = dis * acc + (dis * dis) * hw2_ref[...] + mb_ref[...][None, :]
    ow = ow_ref[...]
    p_ref[...] = jnp.dot(mu, ow[:128, :], preferred_element_type=jnp.float32) + ob_ref[...]
    q_ref[...] = jnp.dot(mu, ow[128:, :], preferred_element_type=jnp.float32)


def _t4(acc2, dis, hw2, mu_b, out_w, out_b):
    return pl.pallas_call(
        _t4_body,
        out_shape=(
            jax.ShapeDtypeStruct((N, 1), jnp.float32),
            jax.ShapeDtypeStruct((N, 1), jnp.float32),
        ),
    )(acc2, dis, hw2, mu_b, out_w, out_b)


_SLOTS = 5
_CP = 40
_CD = 80

_EPT_DEG = E // (NC * NS)
_NCH_DEG = _EPT_DEG // _CD


def _deg_scratch():
    out = []
    for _ in range(_SLOTS):
        out.append(pltpu.VMEM((_CD,), jnp.int32))
    for _ in range(_SLOTS):
        out.append(pltpu.VMEM((_CD,), jnp.float32))
    out.append(pltpu.VMEM_SHARED((NPAD,), jnp.float32))
    out.append(pltpu.SemaphoreType.DMA((_SLOTS,)))
    out.append(pltpu.SemaphoreType.DMA((_SLOTS,)))
    return out


@functools.partial(
    pl.kernel,
    out_type=(
        jax.ShapeDtypeStruct((NPAD,), jnp.float32),
        jax.ShapeDtypeStruct((NPAD,), jnp.float32),
    ),
    mesh=_mesh(),
    scratch_types=_deg_scratch(),
)
def _deg_kernel(dst_hbm, w_hbm, zeros_hbm, zi_hbm, zf_hbm, deg0_hbm, deg1_hbm, *r):
    dsts = r[0:_SLOTS]
    ws = r[_SLOTS:2 * _SLOTS]
    deg_sh, esem, ssem = r[2 * _SLOTS], r[2 * _SLOTS + 1], r[2 * _SLOTS + 2]
    c = lax.axis_index("c")
    s = lax.axis_index("s")
    base = (c * NS + s) * _EPT_DEG
    pltpu.sync_copy(zeros_hbm.at[pl.ds(0, RTILE)], deg_sh.at[pl.ds(s * RTILE, RTILE)])
    plsc.subcore_barrier()

    def el(i, b):
        eb = base + i * _CD
        pltpu.async_copy(dst_hbm.at[pl.ds(eb, _CD)], dsts[b], esem.at[b])
        pltpu.async_copy(w_hbm.at[pl.ds(eb, _CD)], ws[b], esem.at[b])

    def we(i, b):
        eb = base + i * _CD
        pltpu.make_async_copy(dst_hbm.at[pl.ds(eb, _CD)], dsts[b], esem.at[b]).wait()
        pltpu.make_async_copy(w_hbm.at[pl.ds(eb, _CD)], ws[b], esem.at[b]).wait()

    def ss(i, b):
        pltpu.async_copy(ws[b], deg_sh.at[dsts[b]], ssem.at[b], add=True)

    def ws_(i, b):
        pltpu.make_async_copy(ws[b], deg_sh.at[dsts[b]], ssem.at[b]).wait()

    for b in (3, 4):
        pltpu.sync_copy(zi_hbm, dsts[b])
        pltpu.sync_copy(zf_hbm, ws[b])
        ss(-1, b)
    el(0, 0)
    el(1, 1)

    def five(p, carry):
        for b in range(_SLOTS):
            i = p * _SLOTS + b
            we(i, b)
            ss(i, b)
            ws_(i - 2, (b + 3) % _SLOTS)
            el(i + 2, (b + 2) % _SLOTS)
        return carry

    lax.fori_loop(0, _NCH_DEG // _SLOTS, five, 0)
    ws_(_NCH_DEG - 2, (_NCH_DEG - 2) % _SLOTS)
    ws_(_NCH_DEG - 1, (_NCH_DEG - 1) % _SLOTS)
    we(_NCH_DEG, _NCH_DEG % _SLOTS)
    we(_NCH_DEG + 1, (_NCH_DEG + 1) % _SLOTS)
    plsc.subcore_barrier()

    @pl.when((s == 0) & (c == 0))
    def _():
        pltpu.sync_copy(deg_sh, deg0_hbm)

    @pl.when((s == 0) & (c == 1))
    def _():
        pltpu.sync_copy(deg_sh, deg1_hbm)


def _prop_scratch(cp, cpad, bf16_table):
    out = []
    for _ in range(_SLOTS):
        out.append(pltpu.VMEM((cpad,), jnp.int32))
    for _ in range(_SLOTS):
        out.append(pltpu.VMEM((cp,), jnp.int32))
    for _ in range(_SLOTS):
        out.append(pltpu.VMEM((cpad,), jnp.float32))
    for _ in range(_SLOTS):
        out.append(pltpu.VMEM((cp, 128),
                              jnp.bfloat16 if bf16_table else jnp.float32))
    if bf16_table:
        for _ in range(_SLOTS):
            out.append(pltpu.VMEM((cp, 128), jnp.float32))
    out.append(pltpu.VMEM_SHARED((NPAD, 128), jnp.float32))
    out.append(pltpu.SemaphoreType.DMA((_SLOTS,)))
    out.append(pltpu.SemaphoreType.DMA((_SLOTS,)))
    out.append(pltpu.SemaphoreType.DMA((_SLOTS,)))
    return out


def _make_prop(feature_split, cp, bf16_table):
    ept = E // NS if feature_split else E // (NC * NS)
    nch = ept // cp
    cpad = ((cp + 15) // 16) * 16
    tdt = jnp.bfloat16 if bf16_table else jnp.float32
    assert nch % _SLOTS == 0

    @functools.partial(
        pl.kernel,
        out_type=jax.ShapeDtypeStruct((2 * N, 128), jnp.float32),
        mesh=_mesh(),
        scratch_types=_prop_scratch(cp, cpad, bf16_table),
    )
    def prop(src_hbm, dst_hbm, w_hbm, tbl_hbm, zeros_hbm, zi_hbm, out_hbm, *r):
        srcs = r[0:_SLOTS]
        dsts = r[_SLOTS:2 * _SLOTS]
        wbs = r[2 * _SLOTS:3 * _SLOTS]
        rows = r[3 * _SLOTS:4 * _SLOTS]
        if bf16_table:
            outs = r[4 * _SLOTS:5 * _SLOTS]
            k = 5 * _SLOTS
        else:
            outs = rows
            k = 4 * _SLOTS
        acc_sh = r[k]
        esem, gsem, ssem = r[k + 1], r[k + 2], r[k + 3]
        c = lax.axis_index("c")
        s = lax.axis_index("s")
        base = (s * ept) if feature_split else ((c * NS + s) * ept)
        pltpu.sync_copy(zeros_hbm, acc_sh.at[pl.ds(s * RTILE, RTILE)])
        cN = c * N
        plsc.subcore_barrier()

        def el(i, b):
            eb = base + i * cp
            pltpu.async_copy(src_hbm.at[pl.ds(eb, cp)], srcs[b].at[pl.ds(0, cp)],
                             esem.at[b])
            pltpu.async_copy(dst_hbm.at[pl.ds(eb, cp)], dsts[b], esem.at[b])
            pltpu.async_copy(w_hbm.at[pl.ds(eb, cp)], wbs[b].at[pl.ds(0, cp)],
                             esem.at[b])

        def we(i, b):
            eb = base + i * cp
            pltpu.make_async_copy(src_hbm.at[pl.ds(eb, cp)],
                                  srcs[b].at[pl.ds(0, cp)], esem.at[b]).wait()
            pltpu.make_async_copy(dst_hbm.at[pl.ds(eb, cp)], dsts[b],
                                  esem.at[b]).wait()
            pltpu.make_async_copy(w_hbm.at[pl.ds(eb, cp)],
                                  wbs[b].at[pl.ds(0, cp)], esem.at[b]).wait()

        def adj(b):
            if feature_split:
                for j in range(cpad // 16):
                    srcs[b][pl.ds(j * 16, 16)] = srcs[b][pl.ds(j * 16, 16)] + cN

        def sg(i, b):
            pltpu.async_copy(tbl_hbm.at[srcs[b].at[pl.ds(0, cp)]], rows[b],
                             gsem.at[b])

        def wg(i, b):
            pltpu.make_async_copy(tbl_hbm.at[srcs[b].at[pl.ds(0, cp)]], rows[b],
                                  gsem.at[b]).wait()

        def ss(i, b):
            pltpu.async_copy(outs[b], acc_sh.at[dsts[b]], ssem.at[b], add=True)

        def ws_(i, b):
            pltpu.make_async_copy(outs[b], acc_sh.at[dsts[b]], ssem.at[b]).wait()

        def scale(b):
            ngr = (cp + 15) // 16
            for g in range(ngr):
                w16 = wbs[b][pl.ds(g * 16, 16)]
                for lane in range(16 if (g + 1) * 16 <= cp else cp - g * 16):
                    e = g * 16 + lane
                    wv = w16[lane]
                    if bf16_table:
                        for j in range(4):
                            v = rows[b][e, pl.ds(j * 32, 32)].astype(jnp.float32)
                            outs[b][e, pl.ds(j * 32, 32)] = v * wv
                    else:
                        for j in range(8):
                            rows[b][e, pl.ds(j * 16, 16)] = (
                                rows[b][e, pl.ds(j * 16, 16)] * wv)

        for b in (3, 4):
            pltpu.sync_copy(zeros_hbm.at[pl.ds(0, cp)], outs[b])
            pltpu.sync_copy(zi_hbm.at[pl.ds(0, cp)], dsts[b])
            ss(-1, b)
        el(0, 0)
        el(1, 1)
        el(2, 2)
        we(0, 0); adj(0); sg(0, 0)
        we(1, 1); adj(1); sg(1, 1)

        def five(p, carry):
            for b in range(_SLOTS):
                i = p * _SLOTS + b
                b2 = (b + 2) % _SLOTS
                b3 = (b + 3) % _SLOTS
                wg(i, b)
                ws_(i - 2, b3)
                el(i + 3, b3)
                we(i + 2, b2)
                adj(b2)
                sg(i + 2, b2)
                scale(b)
                ss(i, b)
            return carry

        lax.fori_loop(0, nch // _SLOTS, five, 0)
        ws_(nch - 2, (nch - 2) % _SLOTS)
        ws_(nch - 1, (nch - 1) % _SLOTS)
        wg(nch, nch % _SLOTS)
        wg(nch + 1, (nch + 1) % _SLOTS)
        we(nch + 2, (nch + 2) % _SLOTS)
        plsc.subcore_barrier()

        @pl.when(s < NS - 1)
        def _():
            pltpu.sync_copy(acc_sh.at[pl.ds(s * RTILE, RTILE)],
                            out_hbm.at[pl.ds(cN + s * RTILE, RTILE)])

        @pl.when(s == NS - 1)
        def _():
            tail = N - (NS - 1) * RTILE
            pltpu.sync_copy(acc_sh.at[pl.ds((NS - 1) * RTILE, tail)],
                            out_hbm.at[pl.ds(cN + (NS - 1) * RTILE, tail)])

    return prop


_prop1 = _make_prop(True, 40, False)
_prop2 = _make_prop(False, 40, False)

_EPT_HEAD = E // (NC * NS)
_GE_H = 2000


@functools.partial(
    pl.kernel,
    out_type=jax.ShapeDtypeStruct((E,), jnp.float32),
    mesh=_mesh(),
    scratch_types=[
        pltpu.VMEM((N,), jnp.float32),
        pltpu.VMEM((N,), jnp.float32),
        pltpu.VMEM((_GE_H,), jnp.int32),
        pltpu.VMEM((_GE_H,), jnp.int32),
        pltpu.VMEM((_GE_H,), jnp.float32),
    ],
    compiler_params=pltpu.CompilerParams(needs_layout_passes=False),
)
def _head_kernel(src_hbm, dst_hbm, p_hbm, q_hbm, out_hbm,
                 p_v, q_v, src_v, dst_v, out_v):
    c = lax.axis_index("c")
    s = lax.axis_index("s")
    base = (c * NS + s) * _EPT_HEAD
    pltpu.sync_copy(p_hbm, p_v)
    pltpu.sync_copy(q_hbm, q_v)

    def group(g, carry):
        gb = base + g * _GE_H
        pltpu.sync_copy(src_hbm.at[pl.ds(gb, _GE_H)], src_v)
        pltpu.sync_copy(dst_hbm.at[pl.ds(gb, _GE_H)], dst_v)

        def step(i, carry2):
            sidx = src_v[pl.ds(i * 16, 16)]
            didx = dst_v[pl.ds(i * 16, 16)]
            t = plsc.load_gather(p_v, [sidx]) + plsc.load_gather(q_v, [didx])
            out_v[pl.ds(i * 16, 16)] = 1.0 / (1.0 + jnp.exp(-t))
            return carry2

        lax.fori_loop(0, _GE_H // 16, step, 0)
        pltpu.sync_copy(out_v, out_hbm.at[pl.ds(gb, _GE_H)])
        return carry

    lax.fori_loop(0, _EPT_HEAD // _GE_H, group, 0)


def kernel(edge_index, x, weights, lin1_w, lin1_b, conv1_w, conv1_b,
           mu_w, mu_b, ls_w, ls_b, out_w, out_b):
    src = edge_index[0]
    dst = edge_index[1]
    pad = 256
    srcp = jnp.concatenate([src, jnp.zeros((pad,), jnp.int32)])
    dstp = jnp.concatenate([dst, jnp.zeros((pad,), jnp.int32)])
    wp = jnp.concatenate([weights, jnp.zeros((pad,), jnp.float32)])
    zrt = jnp.zeros((RTILE,), jnp.float32)
    z128 = jnp.zeros((RTILE, 128), jnp.float32)
    zi40 = jnp.zeros((_CP,), jnp.int32)
    zi80 = jnp.zeros((_CD,), jnp.int32)
    zf80 = jnp.zeros((_CD,), jnp.float32)

    hw1 = _t1(x, lin1_w, lin1_b, conv1_w)
    deg0, deg1 = _deg_kernel(dstp, wp, zrt, zi80, zf80)
    hws1, dis = _t2(deg0[:N].reshape(N, 1), deg1[:N].reshape(N, 1), hw1)
    acc1 = _prop1(srcp, dstp, wp, hws1.reshape(2 * N, 128), z128, zi40)
    hws2, hw2 = _t3(acc1.reshape(2, N, 128), dis, hw1, conv1_b, mu_w)
    acc2 = _prop2(srcp, dstp, wp, hws2, z128, zi40)
    p, q = _t4(acc2.reshape(2, N, 128), dis, hw2, mu_b, out_w, out_b)
    out = _head_kernel(src, dst, p.reshape(N), q.reshape(N))
    return out
